# baseline (speedup 1.0000x reference)
import functools

import jax
import jax.numpy as jnp
from jax import lax
from jax.experimental import pallas as pl
from jax.experimental.pallas import tpu as pltpu
from jax.experimental.pallas import tpu_sc as plsc


_NC, _NS = 2, 16
_NW = _NC * _NS
_TT = 7
_BC = 128
_ITEMS = _TT * _BC
_PER_W = _ITEMS // _NW
_CHUNK = 1024


def kernel(x, embeddings):
    B, T = x.shape
    V, D = embeddings.shape

    xT = x.T
    fill = ((jnp.arange(6 * B, dtype=jnp.int32) * 7919) % V).reshape(6, B)
    xTp = jnp.concatenate([xT, fill], axis=0)
    idxR = xTp.reshape(_TT, 8, _BC, 128).transpose(0, 2, 1, 3)
    idx4 = (idxR * 4).reshape(-1)

    tableP = jnp.pad(embeddings, ((0, 0), (0, 128 - D))).reshape(4 * V, D)

    mesh = plsc.VectorSubcoreMesh(core_axis_name="c", subcore_axis_name="s")

    @functools.partial(
        pl.kernel,
        mesh=mesh,
        out_type=jax.ShapeDtypeStruct((T, 4, _BC, 8, 128), jnp.float32),
        scratch_types=[
            pltpu.VMEM((2, _CHUNK), jnp.int32),
            pltpu.VMEM((2, _CHUNK, D), jnp.float32),
            pltpu.VMEM((8, D, 128), jnp.float32),
            pltpu.SemaphoreType.DMA,
        ],
        compiler_params=pltpu.CompilerParams(
            use_tc_tiling_on_sc=False, needs_layout_passes=False,
            disable_bounds_checks=True),
    )
    def gather_kernel(table_hbm, idx_hbm, out_hbm, idx_v, rows_v, tbuf, sem):
        wid = lax.axis_index("s") * _NC + lax.axis_index("c")
        item0 = wid * _PER_W

        def start_gather(g, buf):
            pltpu.sync_copy(
                idx_hbm.at[pl.ds((item0 + g) * _CHUNK, _CHUNK)],
                idx_v.at[buf])
            pltpu.async_copy(
                table_hbm.at[idx_v.at[buf]], rows_v.at[buf], sem)

        def wait_gather(buf):
            pltpu.make_async_copy(
                table_hbm.at[pl.ds(0, _CHUNK)], rows_v.at[buf], sem).wait()

        iota16 = lax.broadcasted_iota(jnp.int32, (16,), 0)
        dcols = [jnp.broadcast_to(jnp.int32(d), (16,)) for d in range(D)]

        def process(g, buf):
            item = item0 + g
            tt = item // _BC
            c = item % _BC

            @pl.loop(0, 8)
            def _(grp):
                for s_t in range(8):
                    t = tt * 8 + s_t

                    @pl.when(t < T)
                    def _():
                        ridx = iota16 + (s_t * 128) + grp * 16
                        for d in range(D):
                            vals = plsc.load_gather(
                                rows_v.at[buf], [ridx, dcols[d]])
                            tbuf[s_t, d, pl.ds(grp * 16, 16)] = vals

            for s_t in range(8):
                t = tt * 8 + s_t

                @pl.when(t < T)
                def _():
                    for r in range(4):
                        pltpu.sync_copy(
                            tbuf.at[s_t].at[pl.ds(r * 8, 8)],
                            out_hbm.at[t, r, c])

        start_gather(0, 0)

        @pl.loop(0, _PER_W, step=2)
        def _(g0):
            for b in range(2):
                g = g0 + b

                @pl.when(g + 1 < _PER_W)
                def _():
                    start_gather(g + 1, (b + 1) % 2)

                wait_gather(b)
                process(g, b)

    out5 = gather_kernel(tableP, idx4)
    return out5.transpose(2, 4, 0, 1, 3).reshape(B, T, D)


# speedup vs baseline: 1.5733x; 1.5733x over previous
import functools

import jax
import jax.numpy as jnp
from jax import lax
from jax.experimental import pallas as pl
from jax.experimental.pallas import tpu as pltpu
from jax.experimental.pallas import tpu_sc as plsc


_NC, _NS = 2, 16
_NW = _NC * _NS
_TT = 7
_BC = 128
_ITEMS = _TT * _BC
_PER_W = _ITEMS // _NW
_CHUNK = 1024


def kernel(x, embeddings):
    B, T = x.shape
    V, D = embeddings.shape

    xT = x.T
    fill = ((jnp.arange(6 * B, dtype=jnp.int32) * 7919) % V).reshape(6, B)
    xTp = jnp.concatenate([xT, fill], axis=0)
    idxR = xTp.reshape(_TT, 8, _BC, 128).transpose(0, 2, 1, 3)
    idx4 = (idxR * 4).reshape(-1)

    tableP = jnp.pad(embeddings, ((0, 0), (0, 128 - D))).reshape(4 * V, D)

    mesh = plsc.VectorSubcoreMesh(core_axis_name="c", subcore_axis_name="s")

    @functools.partial(
        pl.kernel,
        mesh=mesh,
        out_type=jax.ShapeDtypeStruct((T, 4, _BC, 8, 128), jnp.float32),
        scratch_types=[
            pltpu.VMEM((2, _CHUNK), jnp.int32),
            pltpu.VMEM((2, _CHUNK, D), jnp.float32),
            pltpu.VMEM((8, D, 128), jnp.float32),
            pltpu.SemaphoreType.DMA,
        ],
        compiler_params=pltpu.CompilerParams(
            use_tc_tiling_on_sc=False, needs_layout_passes=False,
            disable_bounds_checks=True),
    )
    def gather_kernel(table_hbm, idx_hbm, out_hbm, idx_v, rows_v, tbuf, sem):
        wid = lax.axis_index("s") * _NC + lax.axis_index("c")
        item0 = wid * _PER_W

        def start_gather(g, buf):
            pltpu.sync_copy(
                idx_hbm.at[pl.ds((item0 + g) * _CHUNK, _CHUNK)],
                idx_v.at[buf])
            pltpu.async_copy(
                table_hbm.at[idx_v.at[buf]], rows_v.at[buf], sem)

        def wait_gather(buf):
            pltpu.make_async_copy(
                table_hbm.at[pl.ds(0, _CHUNK)], rows_v.at[buf], sem).wait()

        iota16 = lax.broadcasted_iota(jnp.int32, (16,), 0)
        diag = [(jnp.int32(d0) + iota16) % D for d0 in range(D)]

        def process(g, buf):
            item = item0 + g
            tt = item // _BC
            c = item % _BC

            n_jb = jnp.minimum(8, T - tt * 8) * 8

            @pl.loop(0, n_jb)
            def _(jb):
                j0 = jb * 16
                s_t = jb // 8
                l0 = (jb % 8) * 16
                ridx = iota16 + j0
                lidx = iota16 + l0
                tb = tbuf.at[s_t]
                for d0 in range(D):
                    vals = plsc.load_gather(
                        rows_v.at[buf], [ridx, diag[d0]])
                    plsc.store_scatter(tb, [diag[d0], lidx], vals)

            for s_t in range(8):
                t = tt * 8 + s_t

                @pl.when(t < T)
                def _():
                    for r in range(4):
                        pltpu.sync_copy(
                            tbuf.at[s_t].at[pl.ds(r * 8, 8)],
                            out_hbm.at[t, r, c])

        start_gather(0, 0)

        @pl.loop(0, _PER_W, step=2)
        def _(g0):
            for b in range(2):
                g = g0 + b

                @pl.when(g + 1 < _PER_W)
                def _():
                    start_gather(g + 1, (b + 1) % 2)

                wait_gather(b)
                process(g, b)

    out5 = gather_kernel(tableP, idx4)
    return out5.transpose(2, 4, 0, 1, 3).reshape(B, T, D)


# speedup vs baseline: 1.6603x; 1.0553x over previous
import functools

import jax
import jax.numpy as jnp
from jax import lax
from jax.experimental import pallas as pl
from jax.experimental.pallas import tpu as pltpu
from jax.experimental.pallas import tpu_sc as plsc


_NC, _NS = 2, 16
_NW = _NC * _NS
_TT = 7
_BC = 128
_ITEMS = _TT * _BC
_PER_W = _ITEMS // _NW
_CHUNK = 1024


def _tc_pad_transpose(embT, V, D):
    C = 2048

    def body(in_ref, out_ref):
        out_ref[:, 0:D] = in_ref[...].T

    return pl.pallas_call(
        body,
        grid=(pl.cdiv(V, C),),
        in_specs=[pl.BlockSpec((D, C), lambda j: (0, j))],
        out_specs=pl.BlockSpec((C, 128), lambda j: (j, 0)),
        out_shape=jax.ShapeDtypeStruct((V, 128), jnp.float32),
    )(embT)


def kernel(x, embeddings):
    B, T = x.shape
    V, D = embeddings.shape

    xT = x.T
    fill = ((jnp.arange(6 * B, dtype=jnp.int32) * 7919) % V).reshape(6, B)
    xTp = jnp.concatenate([xT, fill], axis=0)
    idxR = xTp.reshape(_TT, 8, _BC, 128).transpose(0, 2, 1, 3)
    idx4 = (idxR * 4).reshape(-1)

    tableP = _tc_pad_transpose(embeddings.T, V, D).reshape(4 * V, D)

    mesh = plsc.VectorSubcoreMesh(core_axis_name="c", subcore_axis_name="s")

    @functools.partial(
        pl.kernel,
        mesh=mesh,
        out_type=jax.ShapeDtypeStruct((T, 4, _BC, 8, 128), jnp.float32),
        scratch_types=[
            pltpu.VMEM((2, _CHUNK), jnp.int32),
            pltpu.VMEM((2, _CHUNK, D), jnp.float32),
            pltpu.VMEM((8, D, 128), jnp.float32),
            pltpu.SemaphoreType.DMA,
        ],
        compiler_params=pltpu.CompilerParams(
            use_tc_tiling_on_sc=False, needs_layout_passes=False,
            disable_bounds_checks=True),
    )
    def gather_kernel(table_hbm, idx_hbm, out_hbm, idx_v, rows_v, tbuf, sem):
        wid = lax.axis_index("s") * _NC + lax.axis_index("c")
        item0 = wid * _PER_W

        def start_gather(g, buf):
            pltpu.sync_copy(
                idx_hbm.at[pl.ds((item0 + g) * _CHUNK, _CHUNK)],
                idx_v.at[buf])
            pltpu.async_copy(
                table_hbm.at[idx_v.at[buf]], rows_v.at[buf], sem)

        def wait_gather(buf):
            pltpu.make_async_copy(
                table_hbm.at[pl.ds(0, _CHUNK)], rows_v.at[buf], sem).wait()

        iota16 = lax.broadcasted_iota(jnp.int32, (16,), 0)
        diag = [(jnp.int32(d0) + iota16) % D for d0 in range(D)]

        def process(g, buf):
            item = item0 + g
            tt = item // _BC
            c = item % _BC

            n_jb = jnp.minimum(8, T - tt * 8) * 8

            @pl.loop(0, n_jb)
            def _(jb):
                j0 = jb * 16
                s_t = jb // 8
                l0 = (jb % 8) * 16
                ridx = iota16 + j0
                lidx = iota16 + l0
                tb = tbuf.at[s_t]
                for d0 in range(D):
                    vals = plsc.load_gather(
                        rows_v.at[buf], [ridx, diag[d0]])
                    plsc.store_scatter(tb, [diag[d0], lidx], vals)

            for s_t in range(8):
                t = tt * 8 + s_t

                @pl.when(t < T)
                def _():
                    for r in range(4):
                        pltpu.sync_copy(
                            tbuf.at[s_t].at[pl.ds(r * 8, 8)],
                            out_hbm.at[t, r, c])

        start_gather(0, 0)

        @pl.loop(0, _PER_W, step=2)
        def _(g0):
            for b in range(2):
                g = g0 + b

                @pl.when(g + 1 < _PER_W)
                def _():
                    start_gather(g + 1, (b + 1) % 2)

                wait_gather(b)
                process(g, b)

    out5 = gather_kernel(tableP, idx4)
    return out5.transpose(2, 4, 0, 1, 3).reshape(B, T, D)


# speedup vs baseline: 2.2237x; 1.3393x over previous
import functools

import jax
import jax.numpy as jnp
from jax import lax
from jax.experimental import pallas as pl
from jax.experimental.pallas import tpu as pltpu
from jax.experimental.pallas import tpu_sc as plsc


_NC, _NS = 2, 16
_NW = _NC * _NS
_TT = 7
_BC = 128
_ITEMS = _TT * _BC
_PER_W = _ITEMS // _NW
_CHUNK = 1024


def _tc_pad_transpose(embT, V, D):
    C = 8192

    def body(in_ref, out_ref):
        out_ref[:, 0:D] = in_ref[...].T

    return pl.pallas_call(
        body,
        grid=(pl.cdiv(V, C),),
        in_specs=[pl.BlockSpec((D, C), lambda j: (0, j))],
        out_specs=pl.BlockSpec((C, 128), lambda j: (j, 0)),
        out_shape=jax.ShapeDtypeStruct((V, 128), jnp.float32),
    )(embT)


def kernel(x, embeddings):
    B, T = x.shape
    V, D = embeddings.shape

    xT = x.T
    fill = ((jnp.arange(6 * B, dtype=jnp.int32) * 7919) % V).reshape(6, B)
    xTp = jnp.concatenate([xT, fill], axis=0)
    idxR = xTp.reshape(_TT, 8, _BC, 128).transpose(0, 2, 1, 3)
    idx4 = (idxR * 4).reshape(-1)

    tableP3 = _tc_pad_transpose(embeddings.T, V, D)
    tableP = tableP3.reshape(4 * tableP3.shape[0], D)

    mesh = plsc.VectorSubcoreMesh(core_axis_name="c", subcore_axis_name="s")

    @functools.partial(
        pl.kernel,
        mesh=mesh,
        out_type=jax.ShapeDtypeStruct((T, 4, _BC, 8, 128), jnp.float32),
        scratch_types=[
            pltpu.VMEM((2, _CHUNK), jnp.int32),
            pltpu.VMEM((2, _CHUNK, D), jnp.float32),
            pltpu.VMEM((8, D, 128), jnp.float32),
            pltpu.SemaphoreType.DMA,
        ],
        compiler_params=pltpu.CompilerParams(
            use_tc_tiling_on_sc=False, needs_layout_passes=False,
            disable_bounds_checks=True),
    )
    def gather_kernel(table_hbm, idx_hbm, out_hbm, idx_v, rows_v, tbuf, sem):
        wid = lax.axis_index("s") * _NC + lax.axis_index("c")
        item0 = wid * _PER_W

        def start_gather(g, buf):
            pltpu.sync_copy(
                idx_hbm.at[pl.ds((item0 + g) * _CHUNK, _CHUNK)],
                idx_v.at[buf])
            pltpu.async_copy(
                table_hbm.at[idx_v.at[buf]], rows_v.at[buf], sem)

        def wait_gather(buf):
            pltpu.make_async_copy(
                table_hbm.at[pl.ds(0, _CHUNK)], rows_v.at[buf], sem).wait()

        iota16 = lax.broadcasted_iota(jnp.int32, (16,), 0)
        diag = [(jnp.int32(d0) + iota16) % D for d0 in range(D)]

        def process(g, buf):
            item = item0 + g
            tt = item // _BC
            c = item % _BC

            n_jb = jnp.minimum(8, T - tt * 8) * 8

            @pl.loop(0, n_jb)
            def _(jb):
                j0 = jb * 16
                s_t = jb // 8
                l0 = (jb % 8) * 16
                ridx = iota16 + j0
                lidx = iota16 + l0
                tb = tbuf.at[s_t]
                for d0 in range(D):
                    vals = plsc.load_gather(
                        rows_v.at[buf], [ridx, diag[d0]])
                    plsc.store_scatter(tb, [diag[d0], lidx], vals)

            for s_t in range(8):
                t = tt * 8 + s_t

                @pl.when(t < T)
                def _():
                    for r in range(4):
                        pltpu.sync_copy(
                            tbuf.at[s_t].at[pl.ds(r * 8, 8)],
                            out_hbm.at[t, r, c])

        start_gather(0, 0)

        @pl.loop(0, _PER_W, step=2)
        def _(g0):
            for b in range(2):
                g = g0 + b

                @pl.when(g + 1 < _PER_W)
                def _():
                    start_gather(g + 1, (b + 1) % 2)

                wait_gather(b)
                process(g, b)

    out5 = gather_kernel(tableP, idx4)
    return out5.transpose(2, 4, 0, 1, 3).reshape(B, T, D)


# speedup vs baseline: 2.3489x; 1.0563x over previous
import functools

import jax
import jax.numpy as jnp
from jax import lax
from jax.experimental import pallas as pl
from jax.experimental.pallas import tpu as pltpu
from jax.experimental.pallas import tpu_sc as plsc


_NC, _NS = 2, 16
_NW = _NC * _NS
_TT = 7
_BC = 128
_ITEMS = _TT * _BC
_PER_W = _ITEMS // _NW
_CHUNK = 1024


def _tc_pad_transpose(embT, V, D):
    C = 16384

    def body(in_ref, out_ref):
        out_ref[:, 0:D] = in_ref[...].T

    return pl.pallas_call(
        body,
        grid=(pl.cdiv(V, C),),
        in_specs=[pl.BlockSpec((D, C), lambda j: (0, j))],
        out_specs=pl.BlockSpec((C, 128), lambda j: (j, 0)),
        out_shape=jax.ShapeDtypeStruct((V, 128), jnp.float32),
    )(embT)


def kernel(x, embeddings):
    B, T = x.shape
    V, D = embeddings.shape

    xT = x.T
    fill = ((jnp.arange(6 * B, dtype=jnp.int32) * 7919) % V).reshape(6, B)
    xTp = jnp.concatenate([xT, fill], axis=0)
    idxR = xTp.reshape(_TT, 8, _BC, 128).transpose(0, 2, 1, 3)
    idx4 = (idxR * 4).reshape(-1)

    tableP3 = _tc_pad_transpose(embeddings.T, V, D)
    tableP = tableP3.reshape(4 * tableP3.shape[0], D)

    mesh = plsc.VectorSubcoreMesh(core_axis_name="c", subcore_axis_name="s")

    @functools.partial(
        pl.kernel,
        mesh=mesh,
        out_type=jax.ShapeDtypeStruct((T, 4, _BC, 8, 128), jnp.float32),
        scratch_types=[
            pltpu.VMEM((2, _CHUNK), jnp.int32),
            pltpu.VMEM((2, _CHUNK, D), jnp.float32),
            pltpu.VMEM((8, D, 128), jnp.float32),
            pltpu.SemaphoreType.DMA,
        ],
        compiler_params=pltpu.CompilerParams(
            use_tc_tiling_on_sc=False, needs_layout_passes=False,
            disable_bounds_checks=True),
    )
    def gather_kernel(table_hbm, idx_hbm, out_hbm, idx_v, rows_v, tbuf, sem):
        wid = lax.axis_index("s") * _NC + lax.axis_index("c")
        item0 = wid * _PER_W

        def start_gather(g, buf):
            pltpu.sync_copy(
                idx_hbm.at[pl.ds((item0 + g) * _CHUNK, _CHUNK)],
                idx_v.at[buf])
            pltpu.async_copy(
                table_hbm.at[idx_v.at[buf]], rows_v.at[buf], sem)

        def wait_gather(buf):
            pltpu.make_async_copy(
                table_hbm.at[pl.ds(0, _CHUNK)], rows_v.at[buf], sem).wait()

        iota16 = lax.broadcasted_iota(jnp.int32, (16,), 0)
        diag = [(jnp.int32(d0) + iota16) % D for d0 in range(D)]

        def process(g, buf):
            item = item0 + g
            tt = item // _BC
            c = item % _BC

            n_jb = jnp.minimum(8, T - tt * 8) * 8

            @pl.loop(0, n_jb)
            def _(jb):
                j0 = jb * 16
                s_t = jb // 8
                l0 = (jb % 8) * 16
                ridx = iota16 + j0
                lidx = iota16 + l0
                tb = tbuf.at[s_t]
                for d0 in range(D):
                    vals = plsc.load_gather(
                        rows_v.at[buf], [ridx, diag[d0]])
                    plsc.store_scatter(tb, [diag[d0], lidx], vals)

            for s_t in range(8):
                t = tt * 8 + s_t

                @pl.when(t < T)
                def _():
                    for r in range(4):
                        pltpu.sync_copy(
                            tbuf.at[s_t].at[pl.ds(r * 8, 8)],
                            out_hbm.at[t, r, c])

        start_gather(0, 0)

        @pl.loop(0, _PER_W, step=2)
        def _(g0):
            for b in range(2):
                g = g0 + b

                @pl.when(g + 1 < _PER_W)
                def _():
                    start_gather(g + 1, (b + 1) % 2)

                wait_gather(b)
                process(g, b)

    out5 = gather_kernel(tableP, idx4)
    return out5.transpose(2, 4, 0, 1, 3).reshape(B, T, D)
